# Initial kernel scaffold; baseline (speedup 1.0000x reference)
import jax
import jax.numpy as jnp
from jax.experimental import pallas as pl


def kernel(x, edge_index, kpi_tensor, W_gcn, b_gcn, W_ih, W_hh, b_ih, b_hh, W_head, b_head):
    raise NotImplementedError("write your pallas kernel here")


# speedup vs baseline: 49.3251x; 49.3251x over previous
import functools

import jax
import jax.numpy as jnp
from jax import lax
from jax.experimental import pallas as pl
from jax.experimental.pallas import tpu as pltpu
from jax.experimental.pallas import tpu_sc as plsc

T = 8
N = 10000
E = 320000
ND = 128
KD = 16
H = 64

NC = 2
NS = 16
TPC = T // NC
EPT = E // NS
CHUNK = 1000
NCHUNK = EPT // CHUNK
ROWS_PT = N // 10

_mesh = plsc.VectorSubcoreMesh(core_axis_name="c", subcore_axis_name="s",
                               num_cores=NC, num_subcores=NS)


@functools.partial(
    pl.kernel,
    out_type=jax.ShapeDtypeStruct((T * N,), jnp.float32),
    mesh=_mesh,
    compiler_params=pltpu.CompilerParams(use_tc_tiling_on_sc=False),
    scratch_types=[
        pltpu.VMEM((CHUNK,), jnp.int32),
        pltpu.VMEM((CHUNK,), jnp.float32),
        pltpu.VMEM_SHARED((N,), jnp.float32),
        pltpu.SemaphoreType.DMA,
    ],
)
def _sc_deg(dst_hbm, ones_hbm, zeros_hbm, deg_hbm, idx_v, ones_v, deg_sh, sem):
    cid = lax.axis_index("c")
    sid = lax.axis_index("s")
    pltpu.sync_copy(ones_hbm, ones_v)
    for tt in range(TPC):
        t = cid * TPC + tt
        @pl.when(sid < 10)
        def _():
            pltpu.sync_copy(zeros_hbm.at[pl.ds(sid * ROWS_PT, ROWS_PT)],
                            deg_sh.at[pl.ds(sid * ROWS_PT, ROWS_PT)])
        plsc.subcore_barrier()
        for ci in range(NCHUNK):
            pltpu.sync_copy(
                dst_hbm.at[pl.ds(t * E + sid * EPT + ci * CHUNK, CHUNK)],
                idx_v)
            pltpu.sync_copy(ones_v, deg_sh.at[idx_v], add=True)
        plsc.subcore_barrier()
        @pl.when(sid < 10)
        def _():
            pltpu.sync_copy(deg_sh.at[pl.ds(sid * ROWS_PT, ROWS_PT)],
                            deg_hbm.at[pl.ds(t * N + sid * ROWS_PT, ROWS_PT)])
        plsc.subcore_barrier()


@functools.partial(
    pl.kernel,
    out_type=jax.ShapeDtypeStruct((T * N, H), jnp.float32),
    mesh=_mesh,
    compiler_params=pltpu.CompilerParams(use_tc_tiling_on_sc=False),
    scratch_types=[
        pltpu.VMEM((CHUNK,), jnp.int32),
        pltpu.VMEM((CHUNK,), jnp.int32),
        pltpu.VMEM((CHUNK, H), jnp.float32),
        pltpu.VMEM_SHARED((N, H), jnp.float32),
        pltpu.SemaphoreType.DMA,
    ],
)
def _sc_scatter(y_hbm, srcg_hbm, dst_hbm, acc_hbm, src_v, dst_v, rows_v,
                acc_sh, sem):
    cid = lax.axis_index("c")
    sid = lax.axis_index("s")
    for tt in range(TPC):
        t = cid * TPC + tt
        @pl.when(sid < 10)
        def _():
            pltpu.sync_copy(y_hbm.at[pl.ds(t * N + sid * ROWS_PT, ROWS_PT)],
                            rows_v)
            pltpu.sync_copy(rows_v, acc_sh.at[pl.ds(sid * ROWS_PT, ROWS_PT)])
        plsc.subcore_barrier()
        for ci in range(NCHUNK):
            base = t * E + sid * EPT + ci * CHUNK
            pltpu.sync_copy(srcg_hbm.at[pl.ds(base, CHUNK)], src_v)
            pltpu.async_copy(y_hbm.at[src_v], rows_v, sem).wait()
            pltpu.sync_copy(dst_hbm.at[pl.ds(base, CHUNK)], dst_v)
            pltpu.sync_copy(rows_v, acc_sh.at[dst_v], add=True)
        plsc.subcore_barrier()
        @pl.when(sid < 10)
        def _():
            pltpu.sync_copy(acc_sh.at[pl.ds(sid * ROWS_PT, ROWS_PT)], rows_v)
            pltpu.sync_copy(rows_v,
                            acc_hbm.at[pl.ds(t * N + sid * ROWS_PT, ROWS_PT)])
        plsc.subcore_barrier()


def _tc_y_body(x_ref, w_ref, deg_ref, y_ref):
    xw = jnp.dot(x_ref[...], w_ref[...], preferred_element_type=jnp.float32)
    dinv = lax.rsqrt(deg_ref[...] + 1.0)
    y_ref[...] = xw * jnp.reshape(dinv, (CHUNK, 1))


def _tc_y(x2, w, deg2):
    return pl.pallas_call(
        _tc_y_body,
        grid=(T * N // CHUNK,),
        in_specs=[
            pl.BlockSpec((CHUNK, ND), lambda i: (i, 0)),
            pl.BlockSpec((ND, H), lambda i: (0, 0)),
            pl.BlockSpec((1, 1, CHUNK), lambda i: (i, 0, 0)),
        ],
        out_specs=pl.BlockSpec((CHUNK, H), lambda i: (i, 0)),
        out_shape=jax.ShapeDtypeStruct((T * N, H), jnp.float32),
    )(x2, w, deg2)


def _tc_emb_body(acc_ref, deg_ref, b_ref, emb_ref):
    j = pl.program_id(1)
    dinv = lax.rsqrt(deg_ref[...] + 1.0)
    vals = jnp.tanh(acc_ref[...] * jnp.reshape(dinv, (CHUNK, 1)) + b_ref[...])
    colsum = jnp.sum(vals, axis=0, keepdims=True).reshape(1, 1, H)

    @pl.when(j == 0)
    def _():
        emb_ref[...] = jnp.zeros_like(emb_ref)

    emb_ref[...] += colsum * (1.0 / N)


def _tc_emb(acc2, deg2, b_gcn2):
    nj = N // CHUNK
    return pl.pallas_call(
        _tc_emb_body,
        grid=(T, nj),
        in_specs=[
            pl.BlockSpec((CHUNK, H), lambda t, j: (t * nj + j, 0)),
            pl.BlockSpec((1, 1, CHUNK), lambda t, j: (t * nj + j, 0, 0)),
            pl.BlockSpec((1, H), lambda t, j: (0, 0)),
        ],
        out_specs=pl.BlockSpec((1, 1, H), lambda t, j: (t, 0, 0)),
        out_shape=jax.ShapeDtypeStruct((T, 1, H), jnp.float32),
    )(acc2, deg2, b_gcn2)


def _tc_lstm_body(emb_ref, kpi_ref, wih_ref, whh_ref, bih_ref, bhh_ref,
                  whead_ref, bhead_ref, out_ref):
    emb = emb_ref[...]
    kpi = kpi_ref[...]
    wih = wih_ref[...]
    whh = whh_ref[...]
    bias = bih_ref[...] + bhh_ref[...]
    h = jnp.zeros((1, H), dtype=jnp.float32)
    c = jnp.zeros((1, H), dtype=jnp.float32)
    for t in range(T):
        xt = jnp.concatenate([emb[t:t + 1, :], kpi[t:t + 1, :]], axis=1)
        gates = (lax.dot_general(xt, wih, (((1,), (1,)), ((), ())),
                                 preferred_element_type=jnp.float32)
                 + lax.dot_general(h, whh, (((1,), (1,)), ((), ())),
                                   preferred_element_type=jnp.float32)
                 + bias)
        i = jax.nn.sigmoid(gates[:, 0 * H:1 * H])
        f = jax.nn.sigmoid(gates[:, 1 * H:2 * H])
        g = jnp.tanh(gates[:, 2 * H:3 * H])
        o = jax.nn.sigmoid(gates[:, 3 * H:4 * H])
        c = f * c + i * g
        h = o * jnp.tanh(c)
    s = jnp.sum(h * whead_ref[...], axis=1, keepdims=True)
    out_ref[...] = s + bhead_ref[...]


def _tc_lstm(emb, kpi2, w_ih, w_hh, b_ih2, b_hh2, w_head, b_head2):
    return pl.pallas_call(
        _tc_lstm_body,
        out_shape=jax.ShapeDtypeStruct((1, 1), jnp.float32),
    )(emb, kpi2, w_ih, w_hh, b_ih2, b_hh2, w_head, b_head2)


def kernel(x, edge_index, kpi_tensor, W_gcn, b_gcn, W_ih, W_hh, b_ih, b_hh,
           W_head, b_head):
    x2 = x.reshape(T * N, ND)
    src = edge_index[:, 0, :]
    dst = edge_index[:, 1, :].reshape(T * E)
    srcg = (src + (jnp.arange(T, dtype=jnp.int32) * N)[:, None]).reshape(T * E)
    ones_c = jnp.ones((CHUNK,), dtype=jnp.float32)
    zeros_n = jnp.zeros((N,), dtype=jnp.float32)

    deg = _sc_deg(dst, ones_c, zeros_n)
    deg2 = deg.reshape(T * N // CHUNK, 1, CHUNK)
    y2 = _tc_y(x2, W_gcn, deg2)
    acc2 = _sc_scatter(y2, srcg, dst)
    emb = _tc_emb(acc2, deg2, b_gcn.reshape(1, H)).reshape(T, H)
    return _tc_lstm(emb, kpi_tensor.reshape(T, KD), W_ih, W_hh,
                    b_ih.reshape(1, 4 * H), b_hh.reshape(1, 4 * H),
                    W_head, b_head.reshape(1, 1))


# speedup vs baseline: 60.1221x; 1.2189x over previous
import functools

import jax
import jax.numpy as jnp
from jax import lax
from jax.experimental import pallas as pl
from jax.experimental.pallas import tpu as pltpu
from jax.experimental.pallas import tpu_sc as plsc

T = 8
N = 10000
E = 320000
ND = 128
KD = 16
H = 64

NC = 2
NS = 16
TPC = T // NC
EPT = E // NS
CHUNK = 1000
NCHUNK = EPT // CHUNK
SCH = 400
NSCH = EPT // SCH
ROWS_PT = N // 10
ROWS16 = N // 16

_mesh = plsc.VectorSubcoreMesh(core_axis_name="c", subcore_axis_name="s",
                               num_cores=NC, num_subcores=NS)


@functools.partial(
    pl.kernel,
    out_type=jax.ShapeDtypeStruct((T * N,), jnp.float32),
    mesh=_mesh,
    compiler_params=pltpu.CompilerParams(use_tc_tiling_on_sc=False),
    scratch_types=[
        pltpu.VMEM((CHUNK,), jnp.int32),
        pltpu.VMEM((CHUNK,), jnp.float32),
        pltpu.VMEM_SHARED((N,), jnp.float32),
        pltpu.SemaphoreType.DMA,
    ],
)
def _sc_deg(dst_hbm, ones_hbm, zeros_hbm, deg_hbm, idx_v, ones_v, deg_sh, sem):
    cid = lax.axis_index("c")
    sid = lax.axis_index("s")
    pltpu.sync_copy(ones_hbm, ones_v)
    for tt in range(TPC):
        t = cid * TPC + tt
        @pl.when(sid < 10)
        def _():
            pltpu.sync_copy(zeros_hbm.at[pl.ds(sid * ROWS_PT, ROWS_PT)],
                            deg_sh.at[pl.ds(sid * ROWS_PT, ROWS_PT)])
        plsc.subcore_barrier()
        for ci in range(NCHUNK):
            pltpu.sync_copy(
                dst_hbm.at[pl.ds(t * E + sid * EPT + ci * CHUNK, CHUNK)],
                idx_v)
            pltpu.sync_copy(ones_v, deg_sh.at[idx_v], add=True)
        plsc.subcore_barrier()
        @pl.when(sid < 10)
        def _():
            pltpu.sync_copy(deg_sh.at[pl.ds(sid * ROWS_PT, ROWS_PT)],
                            deg_hbm.at[pl.ds(t * N + sid * ROWS_PT, ROWS_PT)])
        plsc.subcore_barrier()


@functools.partial(
    pl.kernel,
    out_type=jax.ShapeDtypeStruct((T * N, H), jnp.float32),
    mesh=_mesh,
    compiler_params=pltpu.CompilerParams(use_tc_tiling_on_sc=False),
    scratch_types=[
        pltpu.VMEM((2, SCH), jnp.int32),
        pltpu.VMEM((2, SCH), jnp.int32),
        pltpu.VMEM((SCH, H), jnp.float32),
        pltpu.VMEM((SCH, H), jnp.float32),
        pltpu.VMEM_SHARED((N, H), jnp.float32),
        pltpu.SemaphoreType.DMA,
        pltpu.SemaphoreType.DMA,
        pltpu.SemaphoreType.DMA,
        pltpu.SemaphoreType.DMA,
    ],
)
def _sc_scatter(y_hbm, ei_hbm, acc_hbm, idx_a, idx_b, rows_a, rows_b,
                acc_sh, isem_a, isem_b, gsem_a, gsem_b):
    cid = lax.axis_index("c")
    sid = lax.axis_index("s")
    idx_v = (idx_a, idx_b)
    rows_v = (rows_a, rows_b)
    isem = (isem_a, isem_b)
    gsem = (gsem_a, gsem_b)
    for tt in range(TPC):
        t = cid * TPC + tt

        def ibase(ci):
            return t * E + sid * EPT + ci * SCH

        pltpu.sync_copy(y_hbm.at[pl.ds(t * N + sid * ROWS16, ROWS16)],
                        acc_sh.at[pl.ds(sid * ROWS16, ROWS16)])
        plsc.subcore_barrier()
        pltpu.async_copy(ei_hbm.at[:, pl.ds(ibase(0), SCH)], idx_v[0],
                         isem[0])
        if NSCH > 1:
            pltpu.async_copy(ei_hbm.at[:, pl.ds(ibase(1), SCH)], idx_v[1],
                             isem[1])
        pltpu.make_async_copy(ei_hbm.at[:, pl.ds(ibase(0), SCH)], idx_v[0],
                              isem[0]).wait()
        pltpu.async_copy(y_hbm.at[idx_a.at[0]], rows_v[0], gsem[0])
        for ci in range(NSCH):
            b = ci & 1
            nb = 1 - b
            if ci + 1 < NSCH:
                pltpu.make_async_copy(
                    ei_hbm.at[:, pl.ds(ibase(ci + 1), SCH)], idx_v[nb],
                    isem[nb]).wait()
                if nb:
                    pltpu.async_copy(y_hbm.at[idx_b.at[0]], rows_v[nb],
                                     gsem[nb])
                else:
                    pltpu.async_copy(y_hbm.at[idx_a.at[0]], rows_v[nb],
                                     gsem[nb])
            if b:
                pltpu.make_async_copy(y_hbm.at[idx_b.at[0]], rows_v[b],
                                      gsem[b]).wait()
                pltpu.sync_copy(rows_v[b], acc_sh.at[idx_b.at[1]], add=True)
            else:
                pltpu.make_async_copy(y_hbm.at[idx_a.at[0]], rows_v[b],
                                      gsem[b]).wait()
                pltpu.sync_copy(rows_v[b], acc_sh.at[idx_a.at[1]], add=True)
            if ci + 2 < NSCH:
                pltpu.async_copy(ei_hbm.at[:, pl.ds(ibase(ci + 2), SCH)],
                                 idx_v[b], isem[b])
        plsc.subcore_barrier()
        pltpu.sync_copy(acc_sh.at[pl.ds(sid * ROWS16, ROWS16)],
                        acc_hbm.at[pl.ds(t * N + sid * ROWS16, ROWS16)])
        plsc.subcore_barrier()


def _tc_xw_body(x_ref, w_ref, xw_ref):
    xw_ref[...] = jnp.dot(x_ref[...], w_ref[...],
                          preferred_element_type=jnp.float32)


def _tc_xw(x2, w):
    return pl.pallas_call(
        _tc_xw_body,
        grid=(T * N // CHUNK,),
        in_specs=[
            pl.BlockSpec((CHUNK, ND), lambda i: (i, 0)),
            pl.BlockSpec((ND, H), lambda i: (0, 0)),
        ],
        out_specs=pl.BlockSpec((CHUNK, H), lambda i: (i, 0)),
        out_shape=jax.ShapeDtypeStruct((T * N, H), jnp.float32),
    )(x2, w)


def _tc_scale_body(xw_ref, deg_ref, y_ref):
    dinv = lax.rsqrt(deg_ref[...] + 1.0)
    y_ref[...] = xw_ref[...] * jnp.reshape(dinv, (CHUNK, 1))


def _tc_scale(xw2, deg2):
    return pl.pallas_call(
        _tc_scale_body,
        grid=(T * N // CHUNK,),
        in_specs=[
            pl.BlockSpec((CHUNK, H), lambda i: (i, 0)),
            pl.BlockSpec((1, 1, CHUNK), lambda i: (i, 0, 0)),
        ],
        out_specs=pl.BlockSpec((CHUNK, H), lambda i: (i, 0)),
        out_shape=jax.ShapeDtypeStruct((T * N, H), jnp.float32),
    )(xw2, deg2)


def _tc_emb_body(acc_ref, deg_ref, b_ref, emb_ref):
    j = pl.program_id(1)
    dinv = lax.rsqrt(deg_ref[...] + 1.0)
    vals = jnp.tanh(acc_ref[...] * jnp.reshape(dinv, (CHUNK, 1)) + b_ref[...])
    colsum = jnp.sum(vals, axis=0, keepdims=True).reshape(1, 1, H)

    @pl.when(j == 0)
    def _():
        emb_ref[...] = jnp.zeros_like(emb_ref)

    emb_ref[...] += colsum * (1.0 / N)


def _tc_emb(acc2, deg2, b_gcn2):
    nj = N // CHUNK
    return pl.pallas_call(
        _tc_emb_body,
        grid=(T, nj),
        in_specs=[
            pl.BlockSpec((CHUNK, H), lambda t, j: (t * nj + j, 0)),
            pl.BlockSpec((1, 1, CHUNK), lambda t, j: (t * nj + j, 0, 0)),
            pl.BlockSpec((1, H), lambda t, j: (0, 0)),
        ],
        out_specs=pl.BlockSpec((1, 1, H), lambda t, j: (t, 0, 0)),
        out_shape=jax.ShapeDtypeStruct((T, 1, H), jnp.float32),
    )(acc2, deg2, b_gcn2)


def _tc_lstm_body(emb_ref, kpi_ref, wih_ref, whh_ref, bih_ref, bhh_ref,
                  whead_ref, bhead_ref, out_ref):
    emb = emb_ref[...]
    kpi = kpi_ref[...]
    wih = wih_ref[...]
    whh = whh_ref[...]
    bias = bih_ref[...] + bhh_ref[...]
    h = jnp.zeros((1, H), dtype=jnp.float32)
    c = jnp.zeros((1, H), dtype=jnp.float32)
    for t in range(T):
        xt = jnp.concatenate([emb[t:t + 1, :], kpi[t:t + 1, :]], axis=1)
        gates = (lax.dot_general(xt, wih, (((1,), (1,)), ((), ())),
                                 preferred_element_type=jnp.float32)
                 + lax.dot_general(h, whh, (((1,), (1,)), ((), ())),
                                   preferred_element_type=jnp.float32)
                 + bias)
        i = jax.nn.sigmoid(gates[:, 0 * H:1 * H])
        f = jax.nn.sigmoid(gates[:, 1 * H:2 * H])
        g = jnp.tanh(gates[:, 2 * H:3 * H])
        o = jax.nn.sigmoid(gates[:, 3 * H:4 * H])
        c = f * c + i * g
        h = o * jnp.tanh(c)
    s = jnp.sum(h * whead_ref[...], axis=1, keepdims=True)
    out_ref[...] = s + bhead_ref[...]


def _tc_lstm(emb, kpi2, w_ih, w_hh, b_ih2, b_hh2, w_head, b_head2):
    return pl.pallas_call(
        _tc_lstm_body,
        out_shape=jax.ShapeDtypeStruct((1, 1), jnp.float32),
    )(emb, kpi2, w_ih, w_hh, b_ih2, b_hh2, w_head, b_head2)


def kernel(x, edge_index, kpi_tensor, W_gcn, b_gcn, W_ih, W_hh, b_ih, b_hh,
           W_head, b_head):
    x2 = x.reshape(T * N, ND)
    src = edge_index[:, 0, :]
    dst = edge_index[:, 1, :].reshape(T * E)
    srcg = (src + (jnp.arange(T, dtype=jnp.int32) * N)[:, None]).reshape(T * E)
    ei2 = jnp.stack([srcg, dst])
    ones_c = jnp.ones((CHUNK,), dtype=jnp.float32)
    zeros_n = jnp.zeros((N,), dtype=jnp.float32)

    xw2 = _tc_xw(x2, W_gcn)
    deg = _sc_deg(dst, ones_c, zeros_n)
    deg2 = deg.reshape(T * N // CHUNK, 1, CHUNK)
    y2 = _tc_scale(xw2, deg2)
    acc2 = _sc_scatter(y2, ei2)
    emb = _tc_emb(acc2, deg2, b_gcn.reshape(1, H)).reshape(T, H)
    return _tc_lstm(emb, kpi_tensor.reshape(T, KD), W_ih, W_hh,
                    b_ih.reshape(1, 4 * H), b_hh.reshape(1, 4 * H),
                    W_head, b_head.reshape(1, 1))
